# Initial kernel scaffold; baseline (speedup 1.0000x reference)
import jax
import jax.numpy as jnp
from jax.experimental import pallas as pl


def kernel(x, edge_index_topo, edge_index_geo, Wt1, bt1, Wt2, bt2, Wg1, bg1, Wg2, bg2, Wf, bf):
    raise NotImplementedError("write your pallas kernel here")


# speedup vs baseline: 1.7101x; 1.7101x over previous
import functools

import jax
import jax.numpy as jnp
from jax import lax
from jax.experimental import pallas as pl
from jax.experimental.pallas import tpu as pltpu
from jax.experimental.pallas import tpu_sc as plsc

N = 10000
E = 320000
F = 128

NUM_WORKERS = 32
EPW = E // NUM_WORKERS
GCH = 80

NPT = 320
NPAD = NUM_WORKERS * NPT
CAP = 16384
IC = 2000


def _mesh():
    return plsc.VectorSubcoreMesh(core_axis_name="c", subcore_axis_name="s")


def _wid():
    return lax.axis_index("s") * 2 + lax.axis_index("c")


def _tc_proj_body(x_ref, wt1_ref, bt1_ref, wg1_ref, bg1_ref,
                  pt_ref, qt_ref, pg_ref, qg_ref):
    x = x_ref[...]
    wta = wt1_ref[:F, :]
    wtb = wt1_ref[F:, :]
    wga = wg1_ref[:F, :]
    wgb = wg1_ref[F:, :]
    dot = functools.partial(jnp.dot, preferred_element_type=jnp.float32)
    pt_ref[...] = dot(x, wta - wtb)
    qt_ref[...] = dot(x, wtb) + bt1_ref[...]
    pg_ref[...] = dot(x, wga - wgb)
    qg_ref[...] = dot(x, wgb) + bg1_ref[...]


def _tc_proj(x, Wt1, bt1, Wg1, bg1):
    BN = 1000
    grid = (N // BN,)
    blk = pl.BlockSpec((BN, F), lambda i: (i, 0))
    full2 = pl.BlockSpec((2 * F, F), lambda i: (0, 0))
    bias = pl.BlockSpec((1, F), lambda i: (0, 0))
    out = jax.ShapeDtypeStruct((N, F), jnp.float32)
    return pl.pallas_call(
        _tc_proj_body,
        grid=grid,
        in_specs=[blk, full2, bias, full2, bias],
        out_specs=[blk, blk, blk, blk],
        out_shape=[out, out, out, out],
    )(x, Wt1, bt1.reshape(1, F), Wg1, bg1.reshape(1, F))


def _tc_edge_body(s_ref, w2_ref, b2_ref, h_ref):
    s = jnp.maximum(s_ref[...], 0.0)
    h = jnp.dot(s, w2_ref[...], preferred_element_type=jnp.float32)
    h_ref[...] = jnp.maximum(h + b2_ref[...], 0.0)


def _tc_edge(S, W2, b2):
    BS = 2000
    grid = (E // BS,)
    blk = pl.BlockSpec((BS, F), lambda i: (i, 0))
    return pl.pallas_call(
        _tc_edge_body,
        grid=grid,
        in_specs=[blk,
                  pl.BlockSpec((F, F), lambda i: (0, 0)),
                  pl.BlockSpec((1, F), lambda i: (0, 0))],
        out_specs=blk,
        out_shape=jax.ShapeDtypeStruct((E, F), jnp.float32),
    )(S, W2, b2.reshape(1, F))


def _tc_final_body(pt_ref, pg_ref, wf_ref, bf_ref, o_ref):
    dot = functools.partial(jnp.dot, preferred_element_type=jnp.float32)
    acc = dot(pt_ref[...], wf_ref[:F, :]) + dot(pg_ref[...], wf_ref[F:, :])
    o_ref[...] = jnp.maximum(acc + bf_ref[...], 0.0)


def _tc_final(pt, pg, Wf, bf):
    BN = 1000
    grid = (N // BN,)
    blk = pl.BlockSpec((BN, F), lambda i: (i, 0))
    return pl.pallas_call(
        _tc_final_body,
        grid=grid,
        in_specs=[blk, blk,
                  pl.BlockSpec((2 * F, F), lambda i: (0, 0)),
                  pl.BlockSpec((1, F), lambda i: (0, 0))],
        out_specs=blk,
        out_shape=jax.ShapeDtypeStruct((N, F), jnp.float32),
    )(pt, pg, Wf, bf.reshape(1, F))


def _sc_gather_body(p_hbm, q_hbm, ii_hbm, jj_hbm, s_hbm, iv, jv, buf, sem):
    wid = _wid()
    base = wid * EPW
    pltpu.sync_copy(ii_hbm.at[pl.ds(base, EPW)], iv)
    pltpu.sync_copy(jj_hbm.at[pl.ds(base, EPW)], jv)

    def body(g, carry):
        off = g * GCH
        pltpu.async_copy(p_hbm.at[iv.at[pl.ds(off, GCH)]], buf, sem).wait()
        pltpu.async_copy(q_hbm.at[jv.at[pl.ds(off, GCH)]], buf, sem,
                         add=True).wait()
        pltpu.sync_copy(buf, s_hbm.at[pl.ds(base + off, GCH)])
        return carry

    lax.fori_loop(0, EPW // GCH, body, 0)


def _sc_gather(P, Q, ii, jj):
    k = functools.partial(
        pl.kernel,
        out_type=jax.ShapeDtypeStruct((E, F), jnp.float32),
        mesh=_mesh(),
        compiler_params=pltpu.CompilerParams(needs_layout_passes=False),
        scratch_types=[
            pltpu.VMEM((EPW,), jnp.int32),
            pltpu.VMEM((EPW,), jnp.int32),
            pltpu.VMEM((GCH, F), jnp.float32),
            pltpu.SemaphoreType.DMA,
        ],
    )(_sc_gather_body)
    return k(P, Q, ii, jj)


def _sc_scatter_body(h_hbm, ii_hbm, pool_hbm, idxb, eids, nls, pooled, rows,
                     sem):
    wid = _wid()
    lo = wid * NPT
    hi = lo + NPT
    iota16 = lax.broadcasted_iota(jnp.int32, (16,), 0)
    zeros16i = jnp.zeros((16,), jnp.int32)
    zeros16f = jnp.zeros((16,), jnp.float32)

    def initi(v, c):
        eids[pl.ds(v * 16, 16)] = zeros16i
        return c
    lax.fori_loop(0, CAP // 16, initi, 0)

    def initp(v, c):
        pooled[pl.ds(v * 16, 16)] = zeros16f
        return c
    lax.fori_loop(0, (NPT * F) // 16, initp, 0)

    def chunk(c, cursor):
        pltpu.sync_copy(ii_hbm.at[pl.ds(c * IC, IC)], idxb)

        def vec(v, cur):
            vals = idxb[pl.ds(v * 16, 16)]
            m = (vals >= lo) & (vals < hi)
            eid = c * IC + v * 16 + iota16
            mi = m.astype(jnp.int32)
            pos = cur + plsc.cumsum(mi) - 1
            plsc.store_scatter(eids, [pos], eid, mask=m)
            plsc.store_scatter(nls, [pos], (vals - lo) * F, mask=m)
            return jnp.minimum(cur + jnp.sum(mi), CAP - 16)

        return lax.fori_loop(0, IC // 16, vec, cursor)

    count = lax.fori_loop(0, E // IC, chunk, jnp.int32(0))

    nblk = (count + GCH - 1) // GCH

    def blk(b, carry):
        off = b * GCH
        pltpu.async_copy(h_hbm.at[eids.at[pl.ds(off, GCH)]], rows, sem).wait()
        nvalid = jnp.minimum(count - off, GCH)

        def edge(r, c2):
            nb = nls[pl.ds(off + r, 16)][0]

            def feat(kk, c3):
                hv = rows[r, pl.ds(kk * 16, 16)]
                pv = pooled[pl.ds(nb + kk * 16, 16)]
                pooled[pl.ds(nb + kk * 16, 16)] = jnp.maximum(pv, hv)
                return c3

            return lax.fori_loop(0, F // 16, feat, c2)

        return lax.fori_loop(0, nvalid, edge, carry)

    lax.fori_loop(0, nblk, blk, 0)

    pltpu.sync_copy(pooled, pool_hbm.at[pl.ds(lo * F, NPT * F)])


def _sc_scatter(H, ii):
    k = functools.partial(
        pl.kernel,
        out_type=jax.ShapeDtypeStruct((NPAD * F,), jnp.float32),
        mesh=_mesh(),
        compiler_params=pltpu.CompilerParams(needs_layout_passes=False),
        scratch_types=[
            pltpu.VMEM((IC,), jnp.int32),
            pltpu.VMEM((CAP,), jnp.int32),
            pltpu.VMEM((CAP,), jnp.int32),
            pltpu.VMEM((NPT * F,), jnp.float32),
            pltpu.VMEM((GCH, F), jnp.float32),
            pltpu.SemaphoreType.DMA,
        ],
    )(_sc_scatter_body)
    return k(H, ii).reshape(NPAD, F)


def kernel(x, edge_index_topo, edge_index_geo,
           Wt1, bt1, Wt2, bt2, Wg1, bg1, Wg2, bg2, Wf, bf):
    ii_t = edge_index_topo[0]
    jj_t = edge_index_topo[1]
    ii_g = edge_index_geo[0]
    jj_g = edge_index_geo[1]

    Pt, Qt, Pg, Qg = _tc_proj(x, Wt1, bt1, Wg1, bg1)

    St = _sc_gather(Pt, Qt, ii_t, jj_t)
    Sg = _sc_gather(Pg, Qg, ii_g, jj_g)

    Ht = _tc_edge(St, Wt2, bt2)
    Hg = _tc_edge(Sg, Wg2, bg2)

    pt = _sc_scatter(Ht, ii_t)
    pg = _sc_scatter(Hg, ii_g)

    return _tc_final(pt, pg, Wf, bf)
